# baseline (speedup 1.0000x reference)
import jax
import jax.numpy as jnp
from jax import lax
from jax.experimental import pallas as pl
from jax.experimental.pallas import tpu as pltpu
from jax.experimental.pallas import tpu_sc as plsc

DEPTH = 1000
N = 16384
NC = 2
NS = 16
NW = NC * NS
L = 16
ROWS_PER_W = N // NW
CHUNK = 32
NCHUNK = ROWS_PER_W // CHUNK


def _onehot_body(idx_hbm, zeros_hbm, out_hbm, idx_v, buf0, buf1, sem0, sem1):
    wid = lax.axis_index("s") * NC + lax.axis_index("c")
    base = wid * ROWS_PER_W

    pltpu.sync_copy(idx_hbm.at[pl.ds(base, ROWS_PER_W)], idx_v)
    z0 = pltpu.async_copy(zeros_hbm, buf0, sem0)
    z1 = pltpu.async_copy(zeros_hbm, buf1, sem1)
    z0.wait()
    z1.wait()

    iota = lax.iota(jnp.int32, L)
    ones = jnp.ones((L,), jnp.float32)
    zvec = jnp.zeros((L,), jnp.float32)
    bufs = (buf0, buf1)
    sems = (sem0, sem1)
    copies = [None, None]
    prev_pos = [None, None]

    for c in range(NCHUNK):
        b = c & 1
        buf = bufs[b]
        if copies[b] is not None:
            copies[b].wait()
            for p in prev_pos[b]:
                plsc.store_scatter(buf, [p], zvec)
        pos = []
        for j in range(CHUNK // L):
            cols = idx_v[pl.ds(c * CHUNK + j * L, L)]
            p = cols + (iota * DEPTH + j * L * DEPTH)
            plsc.store_scatter(buf, [p], ones)
            pos.append(p)
        prev_pos[b] = pos
        copies[b] = pltpu.async_copy(
            buf, out_hbm.at[pl.ds((base + c * CHUNK) * DEPTH, CHUNK * DEPTH)],
            sems[b])

    copies[0].wait()
    copies[1].wait()


_mesh = plsc.VectorSubcoreMesh(core_axis_name="c", subcore_axis_name="s")

_onehot = pl.kernel(
    _onehot_body,
    out_type=jax.ShapeDtypeStruct((N * DEPTH,), jnp.float32),
    mesh=_mesh,
    scratch_types=[
        pltpu.VMEM((ROWS_PER_W,), jnp.int32),
        pltpu.VMEM((CHUNK * DEPTH,), jnp.float32),
        pltpu.VMEM((CHUNK * DEPTH,), jnp.float32),
        pltpu.SemaphoreType.DMA,
        pltpu.SemaphoreType.DMA,
    ],
    compiler_params=pltpu.CompilerParams(
        use_tc_tiling_on_sc=False, needs_layout_passes=False),
)


def kernel(inputs):
    idx = inputs.reshape(-1).astype(jnp.int32)
    zeros = jnp.zeros((CHUNK * DEPTH,), jnp.float32)
    return _onehot(idx, zeros).reshape(N, DEPTH)


# speedup vs baseline: 1.0101x; 1.0101x over previous
import jax
import jax.numpy as jnp
from jax import lax
from jax.experimental import pallas as pl
from jax.experimental.pallas import tpu as pltpu
from jax.experimental.pallas import tpu_sc as plsc

DEPTH = 1000
N = 16384
NC = 2
NS = 16
NW = NC * NS
L = 16
ROWS_PER_W = N // NW
CHUNK = 32
NCHUNK = ROWS_PER_W // CHUNK


def _onehot_body(idx_hbm, zeros_hbm, out_hbm, idx_v, buf0, buf1, sem0, sem1):
    wid = lax.axis_index("s") * NC + lax.axis_index("c")
    base = wid * ROWS_PER_W

    pltpu.sync_copy(idx_hbm.at[pl.ds(base, ROWS_PER_W)], idx_v)
    z0 = pltpu.async_copy(zeros_hbm, buf0, sem0)
    z1 = pltpu.async_copy(zeros_hbm, buf1, sem1)
    z0.wait()
    z1.wait()

    iota = lax.iota(jnp.int32, L)
    ones = jnp.ones((L,), jnp.float32)
    zvec = jnp.zeros((L,), jnp.float32)
    bufs = (buf0, buf1)
    sems = (sem0, sem1)
    copies = [None, None]
    prev_pos = [None, None]

    for c in range(NCHUNK):
        b = c & 1
        buf = bufs[b]
        if copies[b] is not None:
            copies[b].wait()
            for rows, cols in prev_pos[b]:
                plsc.store_scatter(buf, [rows, cols], zvec)
        pos = []
        for j in range(CHUNK // L):
            cols = idx_v[pl.ds(c * CHUNK + j * L, L)]
            rows = iota + (j * L)
            plsc.store_scatter(buf, [rows, cols], ones)
            pos.append((rows, cols))
        prev_pos[b] = pos
        copies[b] = pltpu.async_copy(
            buf, out_hbm.at[pl.ds(base + c * CHUNK, CHUNK)], sems[b])

    copies[0].wait()
    copies[1].wait()


_mesh = plsc.VectorSubcoreMesh(core_axis_name="c", subcore_axis_name="s")

_onehot = pl.kernel(
    _onehot_body,
    out_type=jax.ShapeDtypeStruct((N, DEPTH), jnp.float32),
    mesh=_mesh,
    scratch_types=[
        pltpu.VMEM((ROWS_PER_W,), jnp.int32),
        pltpu.VMEM((CHUNK, DEPTH), jnp.float32),
        pltpu.VMEM((CHUNK, DEPTH), jnp.float32),
        pltpu.SemaphoreType.DMA,
        pltpu.SemaphoreType.DMA,
    ],
    compiler_params=pltpu.CompilerParams(
        use_tc_tiling_on_sc=False, needs_layout_passes=False),
)


def kernel(inputs):
    idx = inputs.reshape(-1).astype(jnp.int32)
    zeros = jnp.zeros((CHUNK, DEPTH), jnp.float32)
    return _onehot(idx, zeros)


# speedup vs baseline: 1.5152x; 1.5001x over previous
import jax
import jax.numpy as jnp
from jax import lax
from jax.experimental import pallas as pl
from jax.experimental.pallas import tpu as pltpu
from jax.experimental.pallas import tpu_sc as plsc

DEPTH = 1000
N = 16384
NC = 2
NS = 16
NW = NC * NS
L = 16
ROWS_PER_W = N // NW
CHUNK = 32
NCHUNK = ROWS_PER_W // CHUNK


def _onehot_body(idx_hbm, zeros_hbm, out_hbm, idx_v, buf0, buf1, sem0, sem1):
    wid = lax.axis_index("s") * NC + lax.axis_index("c")
    base = wid * ROWS_PER_W

    pltpu.sync_copy(idx_hbm.at[pl.ds(base, ROWS_PER_W)], idx_v)
    z0 = pltpu.async_copy(zeros_hbm, buf0, sem0)
    z1 = pltpu.async_copy(zeros_hbm, buf1, sem1)
    z0.wait()
    z1.wait()

    iota = lax.iota(jnp.int32, L)
    ones = jnp.ones((L,), jnp.float32)
    zvec = jnp.zeros((L,), jnp.float32)
    bufs = (buf0, buf1)
    sems = (sem0, sem1)
    copies = [None, None]
    prev_pos = [None, None]

    for c in range(NCHUNK):
        b = c & 1
        buf = bufs[b]
        if copies[b] is not None:
            copies[b].wait()
            for rows, cols in prev_pos[b]:
                plsc.store_scatter(buf, [rows, cols], zvec)
        pos = []
        for j in range(CHUNK // L):
            cols = idx_v[pl.ds(c * CHUNK + j * L, L)]
            rows = iota + (j * L)
            plsc.store_scatter(buf, [rows, cols], ones)
            pos.append((rows, cols))
        prev_pos[b] = pos
        copies[b] = pltpu.async_copy(
            buf, out_hbm.at[pl.ds(base + c * CHUNK, CHUNK)], sems[b])

    copies[0].wait()
    copies[1].wait()


_mesh = plsc.VectorSubcoreMesh(core_axis_name="c", subcore_axis_name="s")

_onehot = pl.kernel(
    _onehot_body,
    out_type=jax.ShapeDtypeStruct((N, DEPTH), jnp.float32),
    mesh=_mesh,
    scratch_types=[
        pltpu.VMEM((ROWS_PER_W,), jnp.int32),
        pltpu.VMEM((CHUNK, DEPTH), jnp.float32),
        pltpu.VMEM((CHUNK, DEPTH), jnp.float32),
        pltpu.SemaphoreType.DMA,
        pltpu.SemaphoreType.DMA,
    ],
    compiler_params=pltpu.CompilerParams(
        use_tc_tiling_on_sc=True, needs_layout_passes=False),
)


def kernel(inputs):
    idx = inputs.reshape(-1).astype(jnp.int32)
    zeros = jnp.zeros((CHUNK, DEPTH), jnp.float32)
    return _onehot(idx, zeros)
